# baseline (speedup 1.0000x reference)
import jax
import jax.numpy as jnp
from jax import lax
from jax.experimental import pallas as pl
from jax.experimental.pallas import tpu as pltpu
from jax.experimental.pallas import tpu_sc as plsc

N = 8192
K = 8192
D = 256
VQ_W = 0.25

BLK_I = 2048
BLK_J = 2048
GRID_I = N // BLK_I
GRID_J = K // BLK_J

SC_CORES = 2
SC_SUBCORES = 16
SC_WORKERS = SC_CORES * SC_SUBCORES
ROWS_PER_WORKER = N // SC_WORKERS


def _cnorm_body(x_ref, c_ref):
    xb = x_ref[...]
    c = jnp.sum(xb * xb, axis=1, keepdims=True)
    c_ref[...] = c.reshape(1, BLK_I)


def _cnorm_call(x):
    return pl.pallas_call(
        _cnorm_body,
        grid=(GRID_I,),
        in_specs=[pl.BlockSpec((BLK_I, D), lambda i: (i, 0))],
        out_specs=pl.BlockSpec((1, BLK_I), lambda i: (0, i)),
        out_shape=jax.ShapeDtypeStruct((1, N), jnp.float32),
    )(x)


def _argmin_body(ct_ref, x_ref, w_ref, idx_ref, loss_ref, best_d, best_i, acc):
    i = pl.program_id(0)
    j = pl.program_id(1)
    xb = x_ref[...]
    wb = w_ref[...]

    ct = ct_ref[...]
    b = jnp.sum(wb * wb, axis=1, keepdims=True)
    mt = lax.dot_general(wb, xb, (((1,), (1,)), ((), ())),
                         preferred_element_type=jnp.float32)
    dt = (ct + b) - 2.0 * mt

    lm = jnp.min(dt, axis=0, keepdims=True)
    ii = lax.broadcasted_iota(jnp.int32, (BLK_J, BLK_I), 0)
    li = jnp.min(jnp.where(dt == lm, ii, K), axis=0, keepdims=True)
    gi = j * BLK_J + li

    @pl.when(j == 0)
    def _():
        best_d[...] = lm
        best_i[...] = gi

    @pl.when(j > 0)
    def _():
        upd = lm < best_d[...]
        best_d[...] = jnp.where(upd, lm, best_d[...])
        best_i[...] = jnp.where(upd, gi, best_i[...])

    @pl.when(j == GRID_J - 1)
    def _():
        idx_ref[0, :, :] = best_i[...]
        part = jnp.sum(best_d[...])

        @pl.when(i == 0)
        def _():
            acc[0, 0] = part

        @pl.when(i > 0)
        def _():
            acc[0, 0] += part

        @pl.when(i == GRID_I - 1)
        def _():
            mse = acc[0, 0] / (N * D)
            loss_ref[0, 0] = mse * VQ_W + mse


def _argmin_call(x, W):
    ct = _cnorm_call(x)
    return pl.pallas_call(
        _argmin_body,
        grid=(GRID_I, GRID_J),
        in_specs=[
            pl.BlockSpec((1, BLK_I), lambda i, j: (0, i)),
            pl.BlockSpec((BLK_I, D), lambda i, j: (i, 0)),
            pl.BlockSpec((BLK_J, D), lambda i, j: (j, 0)),
        ],
        out_specs=[
            pl.BlockSpec((1, 1, BLK_I), lambda i, j: (i, 0, 0)),
            pl.BlockSpec(memory_space=pltpu.SMEM),
        ],
        out_shape=[
            jax.ShapeDtypeStruct((GRID_I, 1, BLK_I), jnp.int32),
            jax.ShapeDtypeStruct((1, 1), jnp.float32),
        ],
        scratch_shapes=[
            pltpu.VMEM((1, BLK_I), jnp.float32),
            pltpu.VMEM((1, BLK_I), jnp.int32),
            pltpu.SMEM((1, 1), jnp.float32),
        ],
    )(ct, x, W)


def _gather_body(table_hbm, idx_hbm, out_hbm, idx_v, rows_v, sem):
    wid = lax.axis_index("s") * SC_CORES + lax.axis_index("c")
    base = wid * ROWS_PER_WORKER
    pltpu.sync_copy(idx_hbm.at[pl.ds(base, ROWS_PER_WORKER)], idx_v)
    pltpu.async_copy(table_hbm.at[idx_v], rows_v, sem).wait()
    pltpu.sync_copy(rows_v, out_hbm.at[pl.ds(base, ROWS_PER_WORKER)])


def _gather_call(W, idx):
    return pl.kernel(
        _gather_body,
        mesh=plsc.VectorSubcoreMesh(core_axis_name="c", subcore_axis_name="s"),
        out_type=jax.ShapeDtypeStruct((N, D), jnp.float32),
        scratch_types=[
            pltpu.VMEM((ROWS_PER_WORKER,), jnp.int32),
            pltpu.VMEM((ROWS_PER_WORKER, D), jnp.float32),
            pltpu.SemaphoreType.DMA,
        ],
    )(W, idx)


def kernel(x, W):
    idx3, loss = _argmin_call(x, W)
    idx = idx3.reshape(N)
    q = _gather_call(W, idx)
    return idx, q, loss.reshape(())


# speedup vs baseline: 1.0714x; 1.0714x over previous
import jax
import jax.numpy as jnp
from jax import lax
from jax.experimental import pallas as pl
from jax.experimental.pallas import tpu as pltpu
from jax.experimental.pallas import tpu_sc as plsc

N = 8192
K = 8192
D = 256
VQ_W = 0.25

BLK_I = 2048
BLK_J = 2048
GRID_I = N // BLK_I
GRID_J = K // BLK_J

SC_CORES = 2
SC_SUBCORES = 16
SC_WORKERS = SC_CORES * SC_SUBCORES
ROWS_PER_WORKER = N // SC_WORKERS
GATHER_CHUNKS = 4
CHUNK_ROWS = ROWS_PER_WORKER // GATHER_CHUNKS


def _argmin_body(x_ref, w_ref, iota_ref, idx_ref, loss_ref, best_d, best_i,
                 acc):
    i = pl.program_id(0)
    j = pl.program_id(1)
    xb = x_ref[...]
    wb = w_ref[...]

    c = jnp.sum(xb * xb, axis=1, keepdims=True)
    b = jnp.sum(wb * wb, axis=1)
    m = lax.dot_general(xb, wb, (((1,), (1,)), ((), ())),
                        preferred_element_type=jnp.float32)
    d = (c + b[None, :]) - 2.0 * m

    lm = jnp.min(d, axis=1, keepdims=True)
    fi = iota_ref[...]
    gi = jnp.min(jnp.where(d == lm, fi, jnp.float32(K)),
                 axis=1, keepdims=True)

    @pl.when(j == 0)
    def _():
        best_d[...] = lm
        best_i[...] = gi

    @pl.when(j > 0)
    def _():
        upd = lm < best_d[...]
        best_d[...] = jnp.where(upd, lm, best_d[...])
        best_i[...] = jnp.where(upd, gi, best_i[...])

    @pl.when(j == GRID_J - 1)
    def _():
        idx_ref[0, :, :] = best_i[...].astype(jnp.int32).reshape(1, BLK_I)
        part = jnp.sum(best_d[...])

        @pl.when(i == 0)
        def _():
            acc[0, 0] = part

        @pl.when(i > 0)
        def _():
            acc[0, 0] += part

        @pl.when(i == GRID_I - 1)
        def _():
            mse = acc[0, 0] / (N * D)
            loss_ref[0, 0] = mse * VQ_W + mse


def _argmin_call(x, W):
    return pl.pallas_call(
        _argmin_body,
        grid=(GRID_I, GRID_J),
        in_specs=[
            pl.BlockSpec((BLK_I, D), lambda i, j: (i, 0)),
            pl.BlockSpec((BLK_J, D), lambda i, j: (j, 0)),
            pl.BlockSpec((1, BLK_J), lambda i, j: (0, j)),
        ],
        out_specs=[
            pl.BlockSpec((1, 1, BLK_I), lambda i, j: (i, 0, 0)),
            pl.BlockSpec(memory_space=pltpu.SMEM),
        ],
        out_shape=[
            jax.ShapeDtypeStruct((GRID_I, 1, BLK_I), jnp.int32),
            jax.ShapeDtypeStruct((1, 1), jnp.float32),
        ],
        scratch_shapes=[
            pltpu.VMEM((BLK_I, 1), jnp.float32),
            pltpu.VMEM((BLK_I, 1), jnp.float32),
            pltpu.SMEM((1, 1), jnp.float32),
        ],
    )(x, W, jnp.arange(K, dtype=jnp.float32).reshape(1, K))


def _gather_body(table_hbm, idx_hbm, out_hbm, idx_v, rows_v, gsems, wsem):
    wid = lax.axis_index("s") * SC_CORES + lax.axis_index("c")
    base = wid * ROWS_PER_WORKER
    pltpu.sync_copy(idx_hbm.at[pl.ds(base, ROWS_PER_WORKER)], idx_v)
    copies = []
    for k in range(GATHER_CHUNKS):
        sl = pl.ds(k * CHUNK_ROWS, CHUNK_ROWS)
        copies.append(
            pltpu.async_copy(table_hbm.at[idx_v.at[sl]], rows_v.at[sl],
                             gsems[k]))
    writes = []
    for k in range(GATHER_CHUNKS):
        copies[k].wait()
        sl = pl.ds(k * CHUNK_ROWS, CHUNK_ROWS)
        out_sl = pl.ds(base + k * CHUNK_ROWS, CHUNK_ROWS)
        writes.append(
            pltpu.async_copy(rows_v.at[sl], out_hbm.at[out_sl], wsem))
    for w in writes:
        w.wait()


def _gather_call(W, idx):
    return pl.kernel(
        _gather_body,
        mesh=plsc.VectorSubcoreMesh(core_axis_name="c", subcore_axis_name="s"),
        out_type=jax.ShapeDtypeStruct((N, D), jnp.float32),
        scratch_types=[
            pltpu.VMEM((ROWS_PER_WORKER,), jnp.int32),
            pltpu.VMEM((ROWS_PER_WORKER, D), jnp.float32),
            [pltpu.SemaphoreType.DMA] * GATHER_CHUNKS,
            pltpu.SemaphoreType.DMA,
        ],
    )(W, idx)


def kernel(x, W):
    idx3, loss = _argmin_call(x, W)
    idx = idx3.reshape(N)
    q = _gather_call(W, idx)
    return idx, q, loss.reshape(())


# speedup vs baseline: 1.1341x; 1.0586x over previous
import jax
import jax.numpy as jnp
from jax import lax
from jax.experimental import pallas as pl
from jax.experimental.pallas import tpu as pltpu
from jax.experimental.pallas import tpu_sc as plsc

N = 8192
K = 8192
D = 256
VQ_W = 0.25

BLK_I = 2048
BLK_J = 4096
GRID_I = N // BLK_I
GRID_J = K // BLK_J

SC_CORES = 2
SC_SUBCORES = 16
SC_WORKERS = SC_CORES * SC_SUBCORES
ROWS_PER_WORKER = N // SC_WORKERS
GATHER_CHUNKS = 4
CHUNK_ROWS = ROWS_PER_WORKER // GATHER_CHUNKS


def _argmin_body(x_ref, w_ref, iota_ref, idx_ref, loss_ref, best_d, best_i,
                 acc):
    i = pl.program_id(0)
    j = pl.program_id(1)
    xb = x_ref[...]
    wb = w_ref[...]

    c = jnp.sum(xb * xb, axis=1, keepdims=True)
    b = jnp.sum(wb * wb, axis=1)
    m = lax.dot_general(xb, wb, (((1,), (1,)), ((), ())),
                        preferred_element_type=jnp.float32)
    d = (c + b[None, :]) - 2.0 * m

    lm = jnp.min(d, axis=1, keepdims=True)
    fi = iota_ref[...]
    gi = jnp.min(jnp.where(d == lm, fi, jnp.float32(K)),
                 axis=1, keepdims=True)

    @pl.when(j == 0)
    def _():
        best_d[...] = lm
        best_i[...] = gi

    @pl.when(j > 0)
    def _():
        upd = lm < best_d[...]
        best_d[...] = jnp.where(upd, lm, best_d[...])
        best_i[...] = jnp.where(upd, gi, best_i[...])

    @pl.when(j == GRID_J - 1)
    def _():
        idx_ref[0, :, :] = best_i[...].astype(jnp.int32).reshape(1, BLK_I)
        part = jnp.sum(best_d[...])

        @pl.when(i == 0)
        def _():
            acc[0, 0] = part

        @pl.when(i > 0)
        def _():
            acc[0, 0] += part

        @pl.when(i == GRID_I - 1)
        def _():
            mse = acc[0, 0] / (N * D)
            loss_ref[0, 0] = mse * VQ_W + mse


def _argmin_call(x, W):
    return pl.pallas_call(
        _argmin_body,
        grid=(GRID_I, GRID_J),
        in_specs=[
            pl.BlockSpec((BLK_I, D), lambda i, j: (i, 0)),
            pl.BlockSpec((BLK_J, D), lambda i, j: (j, 0)),
            pl.BlockSpec((1, BLK_J), lambda i, j: (0, j)),
        ],
        out_specs=[
            pl.BlockSpec((1, 1, BLK_I), lambda i, j: (i, 0, 0)),
            pl.BlockSpec(memory_space=pltpu.SMEM),
        ],
        out_shape=[
            jax.ShapeDtypeStruct((GRID_I, 1, BLK_I), jnp.int32),
            jax.ShapeDtypeStruct((1, 1), jnp.float32),
        ],
        scratch_shapes=[
            pltpu.VMEM((BLK_I, 1), jnp.float32),
            pltpu.VMEM((BLK_I, 1), jnp.float32),
            pltpu.SMEM((1, 1), jnp.float32),
        ],
    )(x, W, jnp.arange(K, dtype=jnp.float32).reshape(1, K))


def _gather_body(table_hbm, idx_hbm, out_hbm, idx_v, rows_v, gsems, wsem):
    wid = lax.axis_index("s") * SC_CORES + lax.axis_index("c")
    base = wid * ROWS_PER_WORKER
    pltpu.sync_copy(idx_hbm.at[pl.ds(base, ROWS_PER_WORKER)], idx_v)
    copies = []
    for k in range(GATHER_CHUNKS):
        sl = pl.ds(k * CHUNK_ROWS, CHUNK_ROWS)
        copies.append(
            pltpu.async_copy(table_hbm.at[idx_v.at[sl]], rows_v.at[sl],
                             gsems[k]))
    writes = []
    for k in range(GATHER_CHUNKS):
        copies[k].wait()
        sl = pl.ds(k * CHUNK_ROWS, CHUNK_ROWS)
        out_sl = pl.ds(base + k * CHUNK_ROWS, CHUNK_ROWS)
        writes.append(
            pltpu.async_copy(rows_v.at[sl], out_hbm.at[out_sl], wsem))
    for w in writes:
        w.wait()


def _gather_call(W, idx):
    return pl.kernel(
        _gather_body,
        mesh=plsc.VectorSubcoreMesh(core_axis_name="c", subcore_axis_name="s"),
        out_type=jax.ShapeDtypeStruct((N, D), jnp.float32),
        scratch_types=[
            pltpu.VMEM((ROWS_PER_WORKER,), jnp.int32),
            pltpu.VMEM((ROWS_PER_WORKER, D), jnp.float32),
            [pltpu.SemaphoreType.DMA] * GATHER_CHUNKS,
            pltpu.SemaphoreType.DMA,
        ],
    )(W, idx)


def kernel(x, W):
    idx3, loss = _argmin_call(x, W)
    idx = idx3.reshape(N)
    q = _gather_call(W, idx)
    return idx, q, loss.reshape(())
